# baseline (speedup 1.0000x reference)
import functools

import jax
import jax.numpy as jnp
from jax import lax
from jax.experimental import pallas as pl
from jax.experimental.pallas import tpu as pltpu
from jax.experimental.pallas import tpu_sc as plsc

_B, _N, _F, _M, _H = 4, 4096, 64, 64, 8
_MH = _M * _H
_TR = 512
_S = 0
_NSC = _N - _S
_NW = 32
_RPW = _NSC // _NW
_TILE = 8
_NT = _RPW // _TILE
_NCH = _N // 16


def _phase1_body(a_ref, stats_ref):
    r = pl.program_id(1)
    a = a_ref[0]
    rowsum = jnp.sum(a, axis=1)
    csum = jnp.sum(a, axis=0)

    dblk = a_ref[0, :, pl.ds(r * _TR, _TR)]
    ii = lax.broadcasted_iota(jnp.int32, (_TR, _TR), 0)
    jj = lax.broadcasted_iota(jnp.int32, (_TR, _TR), 1)
    dg = jnp.sum(jnp.where(ii == jj, dblk, 0.0), axis=1)

    @pl.when(r == 0)
    def _():
        stats_ref[0] = jnp.zeros((_N, 8), jnp.float32)

    stats_ref[0, :, 0:1] += csum[:, None]
    stats_ref[0, pl.ds(r * _TR, _TR), 1:2] = rowsum[:, None]
    stats_ref[0, pl.ds(r * _TR, _TR), 2:3] = dg[:, None]


def _sc_pass_body(a_ref, rowpart_ref, colpart_ref, diag_ref,
                  abuf0, abuf1, colacc, rowloc, diagv, sem0, sem1):
    wid = lax.axis_index("c") * 16 + lax.axis_index("s")
    base_row = _S + wid * _RPW
    bufs = (abuf0, abuf1)
    sems = (sem0, sem1)
    iota16 = lax.iota(jnp.int32, 16)
    zero = jnp.zeros((16,), jnp.float32)
    n_steps = _B * _NT

    def start(g, k):
        b = g // _NT
        t = g % _NT
        row0 = base_row + t * _TILE
        return pltpu.async_copy(a_ref.at[b, pl.ds(row0, _TILE), :],
                                bufs[k], sems[k])

    def compute(g, k):
        buf = bufs[k]
        t = g % _NT
        first = t == 0

        def body(j, accs):
            off = pl.multiple_of(j * 16, 16)
            vs = [buf[i, pl.ds(off, 16)] for i in range(_TILE)]
            csum = (((vs[0] + vs[1]) + (vs[2] + vs[3]))
                    + ((vs[4] + vs[5]) + (vs[6] + vs[7])))
            old = colacc[pl.ds(off, 16)]
            colacc[pl.ds(off, 16)] = jnp.where(first, csum, old + csum)
            return tuple(accs[i] + vs[i] for i in range(_TILE))

        accs = lax.fori_loop(0, _NCH, body, (zero,) * _TILE, unroll=4)
        for i in range(_TILE):
            rowloc[t * _TILE + i, :] = accs[i]

        chunk0 = (t % 2) * _TILE
        col0 = pl.multiple_of(base_row + (t - t % 2) * _TILE, 16)
        dvec = zero
        for i in range(_TILE):
            v = buf[i, pl.ds(col0, 16)]
            dvec = dvec + jnp.where(iota16 == (chunk0 + i), v, 0.0)
        return dvec

    def flush(g):
        b = g // _NT
        pltpu.sync_copy(rowloc, rowpart_ref.at[b, pl.ds(wid * _RPW, _RPW)])
        pltpu.sync_copy(colacc, colpart_ref.at[b, wid])
        pltpu.sync_copy(diagv, diag_ref.at[b, wid])

    start(0, 0)

    def step_pair(i, carry):
        g0 = 2 * i
        g1 = 2 * i + 1
        pltpu.make_async_copy(a_ref.at[0, pl.ds(0, _TILE), :],
                              bufs[0], sems[0]).wait()
        start(g1, 1)
        dvec0 = compute(g0, 0)
        pltpu.make_async_copy(a_ref.at[0, pl.ds(0, _TILE), :],
                              bufs[1], sems[1]).wait()

        @pl.when(i + 1 < n_steps // 2)
        def _():
            start(g0 + 2, 0)

        dvec1 = compute(g1, 1)
        doff = pl.multiple_of(((g0 % _NT) // 2) * 16, 16)
        diagv[pl.ds(doff, 16)] = dvec0 + dvec1

        @pl.when(g1 % _NT == _NT - 1)
        def _():
            flush(g1)

        return carry

    lax.fori_loop(0, n_steps // 2, step_pair, 0)


_sc_pass = functools.partial(
    pl.kernel,
    out_type=[
        jax.ShapeDtypeStruct((_B, _NSC, 16), jnp.float32),
        jax.ShapeDtypeStruct((_B, _NW, _N), jnp.float32),
        jax.ShapeDtypeStruct((_B, _NW, _RPW), jnp.float32),
    ],
    mesh=plsc.VectorSubcoreMesh(core_axis_name="c", subcore_axis_name="s"),
    scratch_types=[
        pltpu.VMEM((_TILE, _N), jnp.float32),
        pltpu.VMEM((_TILE, _N), jnp.float32),
        pltpu.VMEM((_N,), jnp.float32),
        pltpu.VMEM((_RPW, 16), jnp.float32),
        pltpu.VMEM((_RPW,), jnp.float32),
        pltpu.SemaphoreType.DMA,
        pltpu.SemaphoreType.DMA,
    ],
)(_sc_pass_body)


def _phase2_body(*refs):
    it = iter(refs)
    x_ref = next(it)
    stats_ref = next(it) if _S > 0 else None
    rowpart_ref = next(it) if _NSC > 0 else None
    colpart_ref = next(it) if _NSC > 0 else None
    diag_ref = next(it) if _NSC > 0 else None
    wn_ref, w2n_ref, wa_ref, sel_ref, binv_ref, psi_ref = it

    inv_n = 1.0 / _N
    x = x_ref[0]
    h1 = jnp.dot(x, wn_ref[...], preferred_element_type=jnp.float32)
    sumx = jnp.sum(x, axis=0, keepdims=True)
    base = jnp.dot(sumx, w2n_ref[...], preferred_element_type=jnp.float32)

    parts_c = []
    parts_r = []
    parts_d = []
    if _S > 0:
        parts_c.append(stats_ref[0, :, 0:1])
        parts_r.append(stats_ref[0, :_S, 1:2])
        parts_d.append(stats_ref[0, :_S, 2:3])
    if _NSC > 0:
        parts_c.append(jnp.sum(colpart_ref[0], axis=0)[:, None])
        parts_r.append(jnp.sum(rowpart_ref[0], axis=1)[:, None])
        parts_d.append(diag_ref[0])
    cl = parts_c[0] + parts_c[1] if len(parts_c) == 2 else parts_c[0]
    rw = jnp.concatenate(parts_r, axis=0) if len(parts_r) == 2 else parts_r[0]
    dg = jnp.concatenate(parts_d, axis=0) if len(parts_d) == 2 else parts_d[0]

    sum_diag = jnp.sum(dg)
    suma = jnp.sum(rw)
    wa = wa_ref[...]
    base = (base + (sum_diag * inv_n) * wa[1:2]
            + (suma * inv_n * inv_n) * wa[4:5] + wa[5:6])
    pernode = (dg * wa[0:1] + (rw * inv_n) * wa[2:3]
               + (cl * inv_n) * wa[3:4])
    hidden = jnp.maximum(h1 + pernode + base, 0.0)
    s = jnp.sum(hidden, axis=0, keepdims=True)
    psi = jnp.dot(s, sel_ref[...], preferred_element_type=jnp.float32) * inv_n
    psi_ref[0, 0, :] = psi[0] + binv_ref[0]


def kernel(X, A, W_eq, b_eq, W_inv, b_inv):
    n = float(_N)
    Wx = W_eq[:, :, :_F, :]
    wav = W_eq[:, :, _F, :]
    Wn = (Wx[:, 0] + (Wx[:, 2] + Wx[:, 3]) * (1.0 / n))
    Wn = jnp.transpose(Wn, (1, 0, 2)).reshape(_F, _MH)
    W2n = (Wx[:, 1] * (1.0 / n) + Wx[:, 4] * (1.0 / (n * n)))
    W2n = jnp.transpose(W2n, (1, 0, 2)).reshape(_F, _MH)
    wa_rows = [wav[:, p].reshape(_MH) for p in range(5)]
    wa_pack = jnp.stack(wa_rows + [b_eq.reshape(_MH),
                                   jnp.zeros((_MH,), jnp.float32),
                                   jnp.zeros((_MH,), jnp.float32)])
    mh_ids = jnp.arange(_MH, dtype=jnp.int32) // _H
    sel = jnp.where(mh_ids[:, None] == jnp.arange(_M, dtype=jnp.int32)[None, :],
                    W_inv.reshape(_MH)[:, None], 0.0)

    inputs = [X]
    in_specs = [pl.BlockSpec((1, _N, _F), lambda b: (b, 0, 0))]
    if _NSC > 0:
        rowpart, colpart, diag = _sc_pass(A)
        diag = diag.reshape(_B, _NSC, 1)

    if _S > 0:
        stats = pl.pallas_call(
            _phase1_body,
            grid=(_B, _S // _TR),
            in_specs=[pl.BlockSpec((1, _TR, _N), lambda b, r: (b, r, 0))],
            out_specs=pl.BlockSpec((1, _N, 8), lambda b, r: (b, 0, 0)),
            out_shape=jax.ShapeDtypeStruct((_B, _N, 8), jnp.float32),
        )(A)
        inputs.append(stats)
        in_specs.append(pl.BlockSpec((1, _N, 8), lambda b: (b, 0, 0)))
    if _NSC > 0:
        inputs += [rowpart, colpart, diag]
        in_specs += [
            pl.BlockSpec((1, _NSC, 16), lambda b: (b, 0, 0)),
            pl.BlockSpec((1, _NW, _N), lambda b: (b, 0, 0)),
            pl.BlockSpec((1, _NSC, 1), lambda b: (b, 0, 0)),
        ]

    inputs += [Wn, W2n, wa_pack, sel, b_inv.reshape(1, _M)]
    in_specs += [
        pl.BlockSpec((_F, _MH), lambda b: (0, 0)),
        pl.BlockSpec((_F, _MH), lambda b: (0, 0)),
        pl.BlockSpec((8, _MH), lambda b: (0, 0)),
        pl.BlockSpec((_MH, _M), lambda b: (0, 0)),
        pl.BlockSpec((1, _M), lambda b: (0, 0)),
    ]
    psi = pl.pallas_call(
        _phase2_body,
        grid=(_B,),
        in_specs=in_specs,
        out_specs=pl.BlockSpec((1, 1, _M), lambda b: (b, 0, 0)),
        out_shape=jax.ShapeDtypeStruct((_B, 1, _M), jnp.float32),
    )(*inputs)
    return psi.reshape(_B, _M)


# speedup vs baseline: 1.8117x; 1.8117x over previous
import functools

import jax
import jax.numpy as jnp
from jax import lax
from jax.experimental import pallas as pl
from jax.experimental.pallas import tpu as pltpu
from jax.experimental.pallas import tpu_sc as plsc

_B, _N, _F, _M, _H = 4, 4096, 64, 64, 8
_MH = _M * _H
_TR = 512
_S = 3072
_NSC = _N - _S
_NW = 32
_RPW = _NSC // _NW
_TILE = 8
_NT = _RPW // _TILE
_NCH = _N // 16


def _phase1_body(a_ref, stats_ref):
    r = pl.program_id(1)
    a = a_ref[0]
    rowsum = jnp.sum(a, axis=1)
    csum = jnp.sum(a, axis=0)

    dblk = a_ref[0, :, pl.ds(r * _TR, _TR)]
    ii = lax.broadcasted_iota(jnp.int32, (_TR, _TR), 0)
    jj = lax.broadcasted_iota(jnp.int32, (_TR, _TR), 1)
    dg = jnp.sum(jnp.where(ii == jj, dblk, 0.0), axis=1)

    @pl.when(r == 0)
    def _():
        stats_ref[0] = jnp.zeros((_N, 8), jnp.float32)

    stats_ref[0, :, 0:1] += csum[:, None]
    stats_ref[0, pl.ds(r * _TR, _TR), 1:2] = rowsum[:, None]
    stats_ref[0, pl.ds(r * _TR, _TR), 2:3] = dg[:, None]


def _sc_pass_body(a_ref, rowpart_ref, colpart_ref, diag_ref,
                  abuf0, abuf1, colacc, rowloc, diagv, sem0, sem1):
    wid = lax.axis_index("c") * 16 + lax.axis_index("s")
    base_row = _S + wid * _RPW
    bufs = (abuf0, abuf1)
    sems = (sem0, sem1)
    iota16 = lax.iota(jnp.int32, 16)
    zero = jnp.zeros((16,), jnp.float32)
    n_steps = _B * _NT

    def start(g, k):
        b = g // _NT
        t = g % _NT
        row0 = base_row + t * _TILE
        return pltpu.async_copy(a_ref.at[b, pl.ds(row0, _TILE), :],
                                bufs[k], sems[k])

    def compute(g, k):
        buf = bufs[k]
        t = g % _NT
        first = t == 0

        def body(j, accs):
            off = pl.multiple_of(j * 16, 16)
            vs = [buf[i, pl.ds(off, 16)] for i in range(_TILE)]
            csum = (((vs[0] + vs[1]) + (vs[2] + vs[3]))
                    + ((vs[4] + vs[5]) + (vs[6] + vs[7])))
            old = colacc[pl.ds(off, 16)]
            colacc[pl.ds(off, 16)] = jnp.where(first, csum, old + csum)
            return tuple(accs[i] + vs[i] for i in range(_TILE))

        accs = lax.fori_loop(0, _NCH, body, (zero,) * _TILE, unroll=4)
        for i in range(_TILE):
            rowloc[t * _TILE + i, :] = accs[i]

        chunk0 = (t % 2) * _TILE
        col0 = pl.multiple_of(base_row + (t - t % 2) * _TILE, 16)
        dvec = zero
        for i in range(_TILE):
            v = buf[i, pl.ds(col0, 16)]
            dvec = dvec + jnp.where(iota16 == (chunk0 + i), v, 0.0)
        return dvec

    def flush(g):
        b = g // _NT
        pltpu.sync_copy(rowloc, rowpart_ref.at[b, pl.ds(wid * _RPW, _RPW)])
        pltpu.sync_copy(colacc, colpart_ref.at[b, wid])
        pltpu.sync_copy(diagv, diag_ref.at[b, wid])

    start(0, 0)

    def step_pair(i, carry):
        g0 = 2 * i
        g1 = 2 * i + 1
        pltpu.make_async_copy(a_ref.at[0, pl.ds(0, _TILE), :],
                              bufs[0], sems[0]).wait()
        start(g1, 1)
        dvec0 = compute(g0, 0)
        pltpu.make_async_copy(a_ref.at[0, pl.ds(0, _TILE), :],
                              bufs[1], sems[1]).wait()

        @pl.when(i + 1 < n_steps // 2)
        def _():
            start(g0 + 2, 0)

        dvec1 = compute(g1, 1)
        doff = pl.multiple_of(((g0 % _NT) // 2) * 16, 16)
        diagv[pl.ds(doff, 16)] = dvec0 + dvec1

        @pl.when(g1 % _NT == _NT - 1)
        def _():
            flush(g1)

        return carry

    lax.fori_loop(0, n_steps // 2, step_pair, 0)


_sc_pass = functools.partial(
    pl.kernel,
    out_type=[
        jax.ShapeDtypeStruct((_B, _NSC, 16), jnp.float32),
        jax.ShapeDtypeStruct((_B, _NW, _N), jnp.float32),
        jax.ShapeDtypeStruct((_B, _NW, _RPW), jnp.float32),
    ],
    mesh=plsc.VectorSubcoreMesh(core_axis_name="c", subcore_axis_name="s"),
    scratch_types=[
        pltpu.VMEM((_TILE, _N), jnp.float32),
        pltpu.VMEM((_TILE, _N), jnp.float32),
        pltpu.VMEM((_N,), jnp.float32),
        pltpu.VMEM((_RPW, 16), jnp.float32),
        pltpu.VMEM((_RPW,), jnp.float32),
        pltpu.SemaphoreType.DMA,
        pltpu.SemaphoreType.DMA,
    ],
)(_sc_pass_body)


def _phase2_body(*refs):
    it = iter(refs)
    x_ref = next(it)
    stats_ref = next(it) if _S > 0 else None
    rowpart_ref = next(it) if _NSC > 0 else None
    colpart_ref = next(it) if _NSC > 0 else None
    diag_ref = next(it) if _NSC > 0 else None
    wn_ref, w2n_ref, wa_ref, sel_ref, binv_ref, psi_ref = it

    inv_n = 1.0 / _N
    x = x_ref[0]
    h1 = jnp.dot(x, wn_ref[...], preferred_element_type=jnp.float32)
    sumx = jnp.sum(x, axis=0, keepdims=True)
    base = jnp.dot(sumx, w2n_ref[...], preferred_element_type=jnp.float32)

    parts_c = []
    parts_r = []
    parts_d = []
    if _S > 0:
        parts_c.append(stats_ref[0, :, 0:1])
        parts_r.append(stats_ref[0, :_S, 1:2])
        parts_d.append(stats_ref[0, :_S, 2:3])
    if _NSC > 0:
        parts_c.append(jnp.sum(colpart_ref[0], axis=0)[:, None])
        parts_r.append(jnp.sum(rowpart_ref[0], axis=1)[:, None])
        parts_d.append(diag_ref[0])
    cl = parts_c[0] + parts_c[1] if len(parts_c) == 2 else parts_c[0]
    rw = jnp.concatenate(parts_r, axis=0) if len(parts_r) == 2 else parts_r[0]
    dg = jnp.concatenate(parts_d, axis=0) if len(parts_d) == 2 else parts_d[0]

    sum_diag = jnp.sum(dg)
    suma = jnp.sum(rw)
    wa = wa_ref[...]
    base = (base + (sum_diag * inv_n) * wa[1:2]
            + (suma * inv_n * inv_n) * wa[4:5] + wa[5:6])
    pernode = (dg * wa[0:1] + (rw * inv_n) * wa[2:3]
               + (cl * inv_n) * wa[3:4])
    hidden = jnp.maximum(h1 + pernode + base, 0.0)
    s = jnp.sum(hidden, axis=0, keepdims=True)
    psi = jnp.dot(s, sel_ref[...], preferred_element_type=jnp.float32) * inv_n
    psi_ref[0, 0, :] = psi[0] + binv_ref[0]


def kernel(X, A, W_eq, b_eq, W_inv, b_inv):
    n = float(_N)
    Wx = W_eq[:, :, :_F, :]
    wav = W_eq[:, :, _F, :]
    Wn = (Wx[:, 0] + (Wx[:, 2] + Wx[:, 3]) * (1.0 / n))
    Wn = jnp.transpose(Wn, (1, 0, 2)).reshape(_F, _MH)
    W2n = (Wx[:, 1] * (1.0 / n) + Wx[:, 4] * (1.0 / (n * n)))
    W2n = jnp.transpose(W2n, (1, 0, 2)).reshape(_F, _MH)
    wa_rows = [wav[:, p].reshape(_MH) for p in range(5)]
    wa_pack = jnp.stack(wa_rows + [b_eq.reshape(_MH),
                                   jnp.zeros((_MH,), jnp.float32),
                                   jnp.zeros((_MH,), jnp.float32)])
    mh_ids = jnp.arange(_MH, dtype=jnp.int32) // _H
    sel = jnp.where(mh_ids[:, None] == jnp.arange(_M, dtype=jnp.int32)[None, :],
                    W_inv.reshape(_MH)[:, None], 0.0)

    inputs = [X]
    in_specs = [pl.BlockSpec((1, _N, _F), lambda b: (b, 0, 0))]
    if _NSC > 0:
        rowpart, colpart, diag = _sc_pass(A)
        diag = diag.reshape(_B, _NSC, 1)

    if _S > 0:
        stats = pl.pallas_call(
            _phase1_body,
            grid=(_B, _S // _TR),
            in_specs=[pl.BlockSpec((1, _TR, _N), lambda b, r: (b, r, 0))],
            out_specs=pl.BlockSpec((1, _N, 8), lambda b, r: (b, 0, 0)),
            out_shape=jax.ShapeDtypeStruct((_B, _N, 8), jnp.float32),
        )(A)
        inputs.append(stats)
        in_specs.append(pl.BlockSpec((1, _N, 8), lambda b: (b, 0, 0)))
    if _NSC > 0:
        inputs += [rowpart, colpart, diag]
        in_specs += [
            pl.BlockSpec((1, _NSC, 16), lambda b: (b, 0, 0)),
            pl.BlockSpec((1, _NW, _N), lambda b: (b, 0, 0)),
            pl.BlockSpec((1, _NSC, 1), lambda b: (b, 0, 0)),
        ]

    inputs += [Wn, W2n, wa_pack, sel, b_inv.reshape(1, _M)]
    in_specs += [
        pl.BlockSpec((_F, _MH), lambda b: (0, 0)),
        pl.BlockSpec((_F, _MH), lambda b: (0, 0)),
        pl.BlockSpec((8, _MH), lambda b: (0, 0)),
        pl.BlockSpec((_MH, _M), lambda b: (0, 0)),
        pl.BlockSpec((1, _M), lambda b: (0, 0)),
    ]
    psi = pl.pallas_call(
        _phase2_body,
        grid=(_B,),
        in_specs=in_specs,
        out_specs=pl.BlockSpec((1, 1, _M), lambda b: (b, 0, 0)),
        out_shape=jax.ShapeDtypeStruct((_B, 1, _M), jnp.float32),
    )(*inputs)
    return psi.reshape(_B, _M)


# speedup vs baseline: 2.2667x; 1.2512x over previous
import jax
import jax.numpy as jnp
from jax import lax
from jax.experimental import pallas as pl
from jax.experimental.pallas import tpu as pltpu

_B, _N, _F, _M, _H = 4, 4096, 64, 64, 8
_MH = _M * _H
_TR = 512
_R = _N // _TR


def _fused_body(a_ref, x_ref, wn_ref, w2n_ref, wa_ref, sel_ref, binv_ref,
                psi_ref, stats_ref):
    r = pl.program_id(1)

    @pl.when(r < _R)
    def _():
        a = a_ref[0]
        rowsum = jnp.sum(a, axis=1)
        csum = jnp.sum(a, axis=0)

        dblk = a_ref[0, :, pl.ds(r * _TR, _TR)]
        ii = lax.broadcasted_iota(jnp.int32, (_TR, _TR), 0)
        jj = lax.broadcasted_iota(jnp.int32, (_TR, _TR), 1)
        dg = jnp.sum(jnp.where(ii == jj, dblk, 0.0), axis=1)

        @pl.when(r == 0)
        def _():
            stats_ref[:, 0:1] = jnp.zeros((_N, 1), jnp.float32)

        stats_ref[:, 0:1] += csum[:, None]
        stats_ref[pl.ds(r * _TR, _TR), 1:2] = rowsum[:, None]
        stats_ref[pl.ds(r * _TR, _TR), 2:3] = dg[:, None]

    @pl.when(r == _R)
    def _():
        inv_n = 1.0 / _N
        x = x_ref[0]
        h1 = jnp.dot(x, wn_ref[...], preferred_element_type=jnp.float32)
        sumx = jnp.sum(x, axis=0, keepdims=True)
        base = jnp.dot(sumx, w2n_ref[...], preferred_element_type=jnp.float32)
        cl = stats_ref[:, 0:1]
        rw = stats_ref[:, 1:2]
        dg = stats_ref[:, 2:3]
        sum_diag = jnp.sum(dg)
        suma = jnp.sum(rw)
        wa = wa_ref[...]
        base = (base + (sum_diag * inv_n) * wa[1:2]
                + (suma * inv_n * inv_n) * wa[4:5] + wa[5:6])
        pernode = (dg * wa[0:1] + (rw * inv_n) * wa[2:3]
                   + (cl * inv_n) * wa[3:4])
        hidden = jnp.maximum(h1 + pernode + base, 0.0)
        s = jnp.sum(hidden, axis=0, keepdims=True)
        psi = jnp.dot(s, sel_ref[...],
                      preferred_element_type=jnp.float32) * inv_n
        psi_ref[0, 0, :] = psi[0] + binv_ref[0]


def kernel(X, A, W_eq, b_eq, W_inv, b_inv):
    n = float(_N)
    Wx = W_eq[:, :, :_F, :]
    wav = W_eq[:, :, _F, :]
    Wn = (Wx[:, 0] + (Wx[:, 2] + Wx[:, 3]) * (1.0 / n))
    Wn = jnp.transpose(Wn, (1, 0, 2)).reshape(_F, _MH)
    W2n = (Wx[:, 1] * (1.0 / n) + Wx[:, 4] * (1.0 / (n * n)))
    W2n = jnp.transpose(W2n, (1, 0, 2)).reshape(_F, _MH)
    wa_rows = [wav[:, p].reshape(_MH) for p in range(5)]
    wa_pack = jnp.stack(wa_rows + [b_eq.reshape(_MH),
                                   jnp.zeros((_MH,), jnp.float32),
                                   jnp.zeros((_MH,), jnp.float32)])
    mh_ids = jnp.arange(_MH, dtype=jnp.int32) // _H
    sel = jnp.where(mh_ids[:, None] == jnp.arange(_M, dtype=jnp.int32)[None, :],
                    W_inv.reshape(_MH)[:, None], 0.0)

    psi = pl.pallas_call(
        _fused_body,
        grid=(_B, _R + 1),
        in_specs=[
            pl.BlockSpec((1, _TR, _N), lambda b, r: (b, jnp.minimum(r, _R - 1), 0)),
            pl.BlockSpec((1, _N, _F), lambda b, r: (b, 0, 0)),
            pl.BlockSpec((_F, _MH), lambda b, r: (0, 0)),
            pl.BlockSpec((_F, _MH), lambda b, r: (0, 0)),
            pl.BlockSpec((8, _MH), lambda b, r: (0, 0)),
            pl.BlockSpec((_MH, _M), lambda b, r: (0, 0)),
            pl.BlockSpec((1, _M), lambda b, r: (0, 0)),
        ],
        out_specs=pl.BlockSpec((1, 1, _M), lambda b, r: (b, 0, 0)),
        out_shape=jax.ShapeDtypeStruct((_B, 1, _M), jnp.float32),
        scratch_shapes=[pltpu.VMEM((_N, 8), jnp.float32)],
    )(A, X, Wn, W2n, wa_pack, sel, b_inv.reshape(1, _M))
    return psi.reshape(_B, _M)


# speedup vs baseline: 2.2783x; 1.0051x over previous
import jax
import jax.numpy as jnp
from jax import lax
from jax.experimental import pallas as pl
from jax.experimental.pallas import tpu as pltpu

_B, _N, _F, _M, _H = 4, 4096, 64, 64, 8
_MH = _M * _H
_TR = 1024
_R = _N // _TR


def _fused_body(a_ref, x_ref, wn_ref, w2n_ref, wa_ref, sel_ref, binv_ref,
                psi_ref, stats_ref):
    r = pl.program_id(1)

    @pl.when(r < _R)
    def _():
        a = a_ref[0]
        rowsum = jnp.sum(a, axis=1)
        csum = jnp.sum(a, axis=0)

        dblk = a_ref[0, :, pl.ds(r * _TR, _TR)]
        ii = lax.broadcasted_iota(jnp.int32, (_TR, _TR), 0)
        jj = lax.broadcasted_iota(jnp.int32, (_TR, _TR), 1)
        dg = jnp.sum(jnp.where(ii == jj, dblk, 0.0), axis=1)

        @pl.when(r == 0)
        def _():
            stats_ref[:, 0:1] = jnp.zeros((_N, 1), jnp.float32)

        stats_ref[:, 0:1] += csum[:, None]
        stats_ref[pl.ds(r * _TR, _TR), 1:2] = rowsum[:, None]
        stats_ref[pl.ds(r * _TR, _TR), 2:3] = dg[:, None]

    @pl.when(r == _R)
    def _():
        inv_n = 1.0 / _N
        x = x_ref[0]
        h1 = jnp.dot(x, wn_ref[...], preferred_element_type=jnp.float32)
        sumx = jnp.sum(x, axis=0, keepdims=True)
        base = jnp.dot(sumx, w2n_ref[...], preferred_element_type=jnp.float32)
        cl = stats_ref[:, 0:1]
        rw = stats_ref[:, 1:2]
        dg = stats_ref[:, 2:3]
        sum_diag = jnp.sum(dg)
        suma = jnp.sum(rw)
        wa = wa_ref[...]
        base = (base + (sum_diag * inv_n) * wa[1:2]
                + (suma * inv_n * inv_n) * wa[4:5] + wa[5:6])
        pernode = (dg * wa[0:1] + (rw * inv_n) * wa[2:3]
                   + (cl * inv_n) * wa[3:4])
        hidden = jnp.maximum(h1 + pernode + base, 0.0)
        s = jnp.sum(hidden, axis=0, keepdims=True)
        psi = jnp.dot(s, sel_ref[...],
                      preferred_element_type=jnp.float32) * inv_n
        psi_ref[0, 0, :] = psi[0] + binv_ref[0]


def kernel(X, A, W_eq, b_eq, W_inv, b_inv):
    n = float(_N)
    Wx = W_eq[:, :, :_F, :]
    wav = W_eq[:, :, _F, :]
    Wn = (Wx[:, 0] + (Wx[:, 2] + Wx[:, 3]) * (1.0 / n))
    Wn = jnp.transpose(Wn, (1, 0, 2)).reshape(_F, _MH)
    W2n = (Wx[:, 1] * (1.0 / n) + Wx[:, 4] * (1.0 / (n * n)))
    W2n = jnp.transpose(W2n, (1, 0, 2)).reshape(_F, _MH)
    wa_rows = [wav[:, p].reshape(_MH) for p in range(5)]
    wa_pack = jnp.stack(wa_rows + [b_eq.reshape(_MH),
                                   jnp.zeros((_MH,), jnp.float32),
                                   jnp.zeros((_MH,), jnp.float32)])
    mh_ids = jnp.arange(_MH, dtype=jnp.int32) // _H
    sel = jnp.where(mh_ids[:, None] == jnp.arange(_M, dtype=jnp.int32)[None, :],
                    W_inv.reshape(_MH)[:, None], 0.0)

    psi = pl.pallas_call(
        _fused_body,
        grid=(_B, _R + 1),
        in_specs=[
            pl.BlockSpec((1, _TR, _N), lambda b, r: (b, jnp.minimum(r, _R - 1), 0)),
            pl.BlockSpec((1, _N, _F), lambda b, r: (b, 0, 0)),
            pl.BlockSpec((_F, _MH), lambda b, r: (0, 0)),
            pl.BlockSpec((_F, _MH), lambda b, r: (0, 0)),
            pl.BlockSpec((8, _MH), lambda b, r: (0, 0)),
            pl.BlockSpec((_MH, _M), lambda b, r: (0, 0)),
            pl.BlockSpec((1, _M), lambda b, r: (0, 0)),
        ],
        out_specs=pl.BlockSpec((1, 1, _M), lambda b, r: (b, 0, 0)),
        out_shape=jax.ShapeDtypeStruct((_B, 1, _M), jnp.float32),
        scratch_shapes=[pltpu.VMEM((_N, 8), jnp.float32)],
    )(A, X, Wn, W2n, wa_pack, sel, b_inv.reshape(1, _M))
    return psi.reshape(_B, _M)
